# baseline (speedup 1.0000x reference)
import functools

import jax
import jax.numpy as jnp
from jax import lax
from jax.experimental import pallas as pl
from jax.experimental.pallas import tpu as pltpu
from jax.experimental.pallas import tpu_sc as plsc

F = 100000
E = 150000
D = 192
L = 16
NC, NS = 2, 16
NW = NC * NS
B = 128
C0, C1 = 38, 36
E_PAD = NS * B * (C0 + C1)
BT = 8192


def _relayout_kernel(xt_ref, o_ref):
    y = xt_ref[...].T.astype(jnp.bfloat16)
    a = y[:, :128]
    b = jnp.concatenate([y[:, 128:], jnp.zeros((BT, 64), jnp.bfloat16)],
                        axis=1)
    au = jax.lax.bitcast_convert_type(a, jnp.uint16).astype(jnp.int32)
    bu = jax.lax.bitcast_convert_type(b, jnp.uint16).astype(jnp.int32)
    o_ref[...] = au | (bu << 16)


def _tv_kernel(patch_hbm, idx0_hbm, idx1_hbm, len_hbm, out_hbm,
               i0a, i1a, lna, f1a, f2a,
               i0b, i1b, lnb, f1b, f2b,
               acc_v, gsema, gsemb, isem):
    cid = lax.axis_index("c")
    sid = lax.axis_index("s")
    phases = ((i0a, i1a, lna, f1a, f2a, gsema),
              (i0b, i1b, lnb, f1b, f2b, gsemb))

    def load_idx(base, c, ph):
        i0, i1, ln, _, _, _ = ph
        off = pl.multiple_of(base + c * B, B)
        cp0 = pltpu.async_copy(idx0_hbm.at[pl.ds(off, B)], i0, isem)
        cp1 = pltpu.async_copy(idx1_hbm.at[pl.ds(off, B)], i1, isem)
        cp2 = pltpu.async_copy(len_hbm.at[pl.ds(off, B)], ln, isem)
        cp0.wait()
        cp1.wait()
        cp2.wait()

    def fire_gathers(patch_hbm, ph):
        i0, i1, _, f1, f2, gsem = ph
        pltpu.async_copy(patch_hbm.at[i0], f1, gsem)
        pltpu.async_copy(patch_hbm.at[i1], f2, gsem)

    def wait_gathers(patch_hbm, ph):
        i0, i1, _, f1, f2, gsem = ph
        pltpu.make_async_copy(patch_hbm.at[i0], f1, gsem).wait()
        pltpu.make_async_copy(patch_hbm.at[i1], f2, gsem).wait()

    def compute(ph, tot):
        _, _, ln, f1, f2, _ = ph

        def group_body(g, t):
            w_blk = ln[pl.ds(g * L, L)]
            for k in range(L):
                e = g * L + k
                facc = None
                for h in range(128 // L):
                    x1 = plsc.bitcast(f1[e, pl.ds(h * L, L)], jnp.bfloat16)
                    x2 = plsc.bitcast(f2[e, pl.ds(h * L, L)], jnp.bfloat16)
                    d = x1 - x2
                    a = jnp.maximum(d, -d)
                    a0, a1 = plsc.unpack(
                        a, format=plsc.PackFormat.INTERLEAVED)
                    facc = a0 + a1 if facc is None else facc + a0 + a1
                t = t + w_blk[k] * facc
            return t

        return lax.fori_loop(0, B // L, group_body, tot)

    def run_core(patch_hbm, nchunks, base):
        base = pl.multiple_of(base, B)
        for b in range(2):
            load_idx(base, b, phases[b])
            fire_gathers(patch_hbm, phases[b])

        def pair_body(c2, tot):
            for b in range(2):
                ph = phases[b]
                c = 2 * c2 + b
                wait_gathers(patch_hbm, ph)
                tot = compute(ph, tot)

                @pl.when(c + 2 < nchunks)
                def _():
                    load_idx(base, c + 2, ph)
                    fire_gathers(patch_hbm, ph)

            return tot

        tot = lax.fori_loop(0, nchunks // 2, pair_body,
                            jnp.zeros((L,), jnp.float32))
        acc_v[...] = tot

    @pl.when(cid == 0)
    def _():
        run_core(patch_hbm, C0, sid * (B * C0))

    @pl.when(cid == 1)
    def _():
        run_core(patch_hbm, C1, NS * (B * C0) + sid * (B * C1))

    wid = sid * NC + cid
    pltpu.sync_copy(acc_v, out_hbm.at[pl.ds(wid * L, L)])


@jax.jit
def kernel(adv_patch, face_to_edges_idx, edge_len):
    patch_t = adv_patch.reshape(F, D).T
    patch_p = pl.pallas_call(
        _relayout_kernel,
        grid=(pl.cdiv(F, BT),),
        compiler_params=pltpu.CompilerParams(
            dimension_semantics=("parallel",)),
        in_specs=[pl.BlockSpec((D, BT), lambda i: (0, i))],
        out_specs=pl.BlockSpec((BT, 128), lambda i: (i, 0)),
        out_shape=jax.ShapeDtypeStruct((F, 128), jnp.int32),
    )(patch_t)

    idx = face_to_edges_idx.astype(jnp.int32)
    pad = E_PAD - E
    fill = jnp.arange(pad, dtype=jnp.int32)
    idx0 = jnp.concatenate([idx[:, 0], fill])
    idx1 = jnp.concatenate([idx[:, 1], fill])
    len_p = jnp.pad(edge_len, (0, pad))

    mesh = plsc.VectorSubcoreMesh(core_axis_name="c", subcore_axis_name="s")
    run = pl.kernel(
        _tv_kernel,
        mesh=mesh,
        compiler_params=pltpu.CompilerParams(use_tc_tiling_on_sc=True,
                                             needs_layout_passes=False),
        out_type=jax.ShapeDtypeStruct((NW * L,), jnp.float32),
        scratch_types=[
            pltpu.VMEM((B,), jnp.int32),
            pltpu.VMEM((B,), jnp.int32),
            pltpu.VMEM((B,), jnp.float32),
            pltpu.VMEM((B, 128), jnp.int32),
            pltpu.VMEM((B, 128), jnp.int32),
            pltpu.VMEM((B,), jnp.int32),
            pltpu.VMEM((B,), jnp.int32),
            pltpu.VMEM((B,), jnp.float32),
            pltpu.VMEM((B, 128), jnp.int32),
            pltpu.VMEM((B, 128), jnp.int32),
            pltpu.VMEM((L,), jnp.float32),
            pltpu.SemaphoreType.DMA,
            pltpu.SemaphoreType.DMA,
            pltpu.SemaphoreType.DMA,
        ],
    )
    partials = run(patch_p, idx0, idx1, len_p)
    return jnp.sum(partials) / F


# speedup vs baseline: 1.0005x; 1.0005x over previous
import functools

import jax
import jax.numpy as jnp
from jax import lax
from jax.experimental import pallas as pl
from jax.experimental.pallas import tpu as pltpu
from jax.experimental.pallas import tpu_sc as plsc

F = 100000
E = 150000
D = 192
L = 16
NC, NS = 2, 16
NW = NC * NS
B = 112
C0, C1 = 42, 42
E_PAD = NS * B * (C0 + C1)
BT = 8192


def _relayout_kernel(xt_ref, o_ref):
    y = xt_ref[...].T.astype(jnp.bfloat16)
    a = y[:, :128]
    b = jnp.concatenate([y[:, 128:], jnp.zeros((BT, 64), jnp.bfloat16)],
                        axis=1)
    au = jax.lax.bitcast_convert_type(a, jnp.uint16).astype(jnp.int32)
    bu = jax.lax.bitcast_convert_type(b, jnp.uint16).astype(jnp.int32)
    o_ref[...] = au | (bu << 16)


def _tv_kernel(patch_hbm, idx0_hbm, idx1_hbm, len_hbm, out_hbm,
               i0a, i1a, lna, f1a, f2a,
               i0b, i1b, lnb, f1b, f2b,
               acc_v, gsema, gsemb, isem):
    cid = lax.axis_index("c")
    sid = lax.axis_index("s")
    phases = ((i0a, i1a, lna, f1a, f2a, gsema),
              (i0b, i1b, lnb, f1b, f2b, gsemb))

    def load_idx(base, c, ph):
        i0, i1, ln, _, _, _ = ph
        off = pl.multiple_of(base + c * B, B)
        cp0 = pltpu.async_copy(idx0_hbm.at[pl.ds(off, B)], i0, isem)
        cp1 = pltpu.async_copy(idx1_hbm.at[pl.ds(off, B)], i1, isem)
        cp2 = pltpu.async_copy(len_hbm.at[pl.ds(off, B)], ln, isem)
        cp0.wait()
        cp1.wait()
        cp2.wait()

    def fire_gathers(patch_hbm, ph):
        i0, i1, _, f1, f2, gsem = ph
        pltpu.async_copy(patch_hbm.at[i0], f1, gsem)
        pltpu.async_copy(patch_hbm.at[i1], f2, gsem)

    def wait_gathers(patch_hbm, ph):
        i0, i1, _, f1, f2, gsem = ph
        pltpu.make_async_copy(patch_hbm.at[i0], f1, gsem).wait()
        pltpu.make_async_copy(patch_hbm.at[i1], f2, gsem).wait()

    def compute(ph, tot):
        _, _, ln, f1, f2, _ = ph

        def group_body(g, t):
            w_blk = ln[pl.ds(g * L, L)]
            for k in range(L):
                e = g * L + k
                facc = None
                for h in range(128 // L):
                    x1 = plsc.bitcast(f1[e, pl.ds(h * L, L)], jnp.bfloat16)
                    x2 = plsc.bitcast(f2[e, pl.ds(h * L, L)], jnp.bfloat16)
                    d = x1 - x2
                    a = jnp.maximum(d, -d)
                    a0, a1 = plsc.unpack(
                        a, format=plsc.PackFormat.INTERLEAVED)
                    facc = a0 + a1 if facc is None else facc + a0 + a1
                t = t + w_blk[k] * facc
            return t

        return lax.fori_loop(0, B // L, group_body, tot)

    def run_core(patch_hbm, nchunks, base):
        base = pl.multiple_of(base, B)
        for b in range(2):
            load_idx(base, b, phases[b])
            fire_gathers(patch_hbm, phases[b])

        def pair_body(c2, tot):
            for b in range(2):
                ph = phases[b]
                c = 2 * c2 + b
                wait_gathers(patch_hbm, ph)
                tot = compute(ph, tot)

                @pl.when(c + 2 < nchunks)
                def _():
                    load_idx(base, c + 2, ph)
                    fire_gathers(patch_hbm, ph)

            return tot

        tot = lax.fori_loop(0, nchunks // 2, pair_body,
                            jnp.zeros((L,), jnp.float32))
        acc_v[...] = tot

    @pl.when(cid == 0)
    def _():
        run_core(patch_hbm, C0, sid * (B * C0))

    @pl.when(cid == 1)
    def _():
        run_core(patch_hbm, C1, NS * (B * C0) + sid * (B * C1))

    wid = sid * NC + cid
    pltpu.sync_copy(acc_v, out_hbm.at[pl.ds(wid * L, L)])


@jax.jit
def kernel(adv_patch, face_to_edges_idx, edge_len):
    patch_t = adv_patch.reshape(F, D).T
    patch_p = pl.pallas_call(
        _relayout_kernel,
        grid=(pl.cdiv(F, BT),),
        compiler_params=pltpu.CompilerParams(
            dimension_semantics=("parallel",)),
        in_specs=[pl.BlockSpec((D, BT), lambda i: (0, i))],
        out_specs=pl.BlockSpec((BT, 128), lambda i: (i, 0)),
        out_shape=jax.ShapeDtypeStruct((F, 128), jnp.int32),
    )(patch_t)

    idx = face_to_edges_idx.astype(jnp.int32)
    pad = E_PAD - E
    fill = jnp.arange(pad, dtype=jnp.int32)
    idx0 = jnp.concatenate([idx[:, 0], fill])
    idx1 = jnp.concatenate([idx[:, 1], fill])
    len_p = jnp.pad(edge_len, (0, pad))

    mesh = plsc.VectorSubcoreMesh(core_axis_name="c", subcore_axis_name="s")
    run = pl.kernel(
        _tv_kernel,
        mesh=mesh,
        compiler_params=pltpu.CompilerParams(use_tc_tiling_on_sc=True,
                                             needs_layout_passes=False),
        out_type=jax.ShapeDtypeStruct((NW * L,), jnp.float32),
        scratch_types=[
            pltpu.VMEM((B,), jnp.int32),
            pltpu.VMEM((B,), jnp.int32),
            pltpu.VMEM((B,), jnp.float32),
            pltpu.VMEM((B, 128), jnp.int32),
            pltpu.VMEM((B, 128), jnp.int32),
            pltpu.VMEM((B,), jnp.int32),
            pltpu.VMEM((B,), jnp.int32),
            pltpu.VMEM((B,), jnp.float32),
            pltpu.VMEM((B, 128), jnp.int32),
            pltpu.VMEM((B, 128), jnp.int32),
            pltpu.VMEM((L,), jnp.float32),
            pltpu.SemaphoreType.DMA,
            pltpu.SemaphoreType.DMA,
            pltpu.SemaphoreType.DMA,
        ],
    )
    partials = run(patch_p, idx0, idx1, len_p)
    return jnp.sum(partials) / F


# speedup vs baseline: 1.0399x; 1.0394x over previous
import functools

import jax
import jax.numpy as jnp
from jax import lax
from jax.experimental import pallas as pl
from jax.experimental.pallas import tpu as pltpu
from jax.experimental.pallas import tpu_sc as plsc

F = 100000
E = 150000
D = 192
L = 16
NC, NS = 2, 16
NW = NC * NS
B = 128
C0, C1 = 38, 36
E_PAD = NS * B * (C0 + C1)
BT = 16384


def _relayout_kernel(xt_ref, o_ref):
    y = xt_ref[...].T.astype(jnp.bfloat16)
    a = y[:, :128]
    b = jnp.concatenate([y[:, 128:], jnp.zeros((BT, 64), jnp.bfloat16)],
                        axis=1)
    au = jax.lax.bitcast_convert_type(a, jnp.uint16).astype(jnp.int32)
    bu = jax.lax.bitcast_convert_type(b, jnp.uint16).astype(jnp.int32)
    o_ref[...] = au | (bu << 16)


def _tv_kernel(patch_hbm, idx01_hbm, len_hbm, out_hbm,
               i01a, lna, f1a, f2a,
               i01b, lnb, f1b, f2b,
               acc_v, gsema, gsemb, isem):
    cid = lax.axis_index("c")
    sid = lax.axis_index("s")
    phases = ((i01a, lna, f1a, f2a, gsema),
              (i01b, lnb, f1b, f2b, gsemb))

    def load_idx(base, c, ph):
        i01, ln, _, _, _ = ph
        off = pl.multiple_of(base + c * B, B)
        cp0 = pltpu.async_copy(idx01_hbm.at[:, pl.ds(off, B)], i01, isem)
        cp1 = pltpu.async_copy(len_hbm.at[pl.ds(off, B)], ln, isem)
        cp0.wait()
        cp1.wait()

    def fire_gathers(patch_hbm, ph):
        i01, _, f1, f2, gsem = ph
        pltpu.async_copy(patch_hbm.at[i01.at[0]], f1, gsem)
        pltpu.async_copy(patch_hbm.at[i01.at[1]], f2, gsem)

    def wait_gathers(patch_hbm, ph):
        i01, _, f1, f2, gsem = ph
        pltpu.make_async_copy(patch_hbm.at[i01.at[0]], f1, gsem).wait()
        pltpu.make_async_copy(patch_hbm.at[i01.at[1]], f2, gsem).wait()

    def compute(ph, tot):
        _, ln, f1, f2, _ = ph

        def group_body(g, t):
            w_blk = ln[pl.ds(g * L, L)]
            for k in range(L):
                e = g * L + k
                facc = None
                for h in range(128 // L):
                    x1 = plsc.bitcast(f1[e, pl.ds(h * L, L)], jnp.bfloat16)
                    x2 = plsc.bitcast(f2[e, pl.ds(h * L, L)], jnp.bfloat16)
                    d = x1 - x2
                    a = jnp.maximum(d, -d)
                    a0, a1 = plsc.unpack(
                        a, format=plsc.PackFormat.INTERLEAVED)
                    facc = a0 + a1 if facc is None else facc + a0 + a1
                t = t + w_blk[k] * facc
            return t

        return lax.fori_loop(0, B // L, group_body, tot)

    def run_core(patch_hbm, nchunks, base):
        base = pl.multiple_of(base, B)
        for b in range(2):
            load_idx(base, b, phases[b])
            fire_gathers(patch_hbm, phases[b])

        def pair_body(c2, tot):
            for b in range(2):
                ph = phases[b]
                c = 2 * c2 + b
                wait_gathers(patch_hbm, ph)
                tot = compute(ph, tot)

                @pl.when(c + 2 < nchunks)
                def _():
                    load_idx(base, c + 2, ph)
                    fire_gathers(patch_hbm, ph)

            return tot

        tot = lax.fori_loop(0, nchunks // 2, pair_body,
                            jnp.zeros((L,), jnp.float32))
        acc_v[...] = tot

    @pl.when(cid == 0)
    def _():
        run_core(patch_hbm, C0, sid * (B * C0))

    @pl.when(cid == 1)
    def _():
        run_core(patch_hbm, C1, NS * (B * C0) + sid * (B * C1))

    wid = sid * NC + cid
    pltpu.sync_copy(acc_v, out_hbm.at[pl.ds(wid * L, L)])


@jax.jit
def kernel(adv_patch, face_to_edges_idx, edge_len):
    patch_t = adv_patch.reshape(F, D).T
    patch_p = pl.pallas_call(
        _relayout_kernel,
        grid=(pl.cdiv(F, BT),),
        compiler_params=pltpu.CompilerParams(
            dimension_semantics=("parallel",)),
        in_specs=[pl.BlockSpec((D, BT), lambda i: (0, i))],
        out_specs=pl.BlockSpec((BT, 128), lambda i: (i, 0)),
        out_shape=jax.ShapeDtypeStruct((F, 128), jnp.int32),
    )(patch_t)

    idx_t = face_to_edges_idx.astype(jnp.int32).T
    pad = E_PAD - E
    fill = jnp.arange(pad, dtype=jnp.int32)
    idx01 = jnp.concatenate(
        [idx_t, jnp.stack([fill, fill])], axis=1)
    len_p = jnp.pad(edge_len, (0, pad))

    mesh = plsc.VectorSubcoreMesh(core_axis_name="c", subcore_axis_name="s")
    run = pl.kernel(
        _tv_kernel,
        mesh=mesh,
        compiler_params=pltpu.CompilerParams(use_tc_tiling_on_sc=True,
                                             needs_layout_passes=False),
        out_type=jax.ShapeDtypeStruct((NW * L,), jnp.float32),
        scratch_types=[
            pltpu.VMEM((2, B), jnp.int32),
            pltpu.VMEM((B,), jnp.float32),
            pltpu.VMEM((B, 128), jnp.int32),
            pltpu.VMEM((B, 128), jnp.int32),
            pltpu.VMEM((2, B), jnp.int32),
            pltpu.VMEM((B,), jnp.float32),
            pltpu.VMEM((B, 128), jnp.int32),
            pltpu.VMEM((B, 128), jnp.int32),
            pltpu.VMEM((L,), jnp.float32),
            pltpu.SemaphoreType.DMA,
            pltpu.SemaphoreType.DMA,
            pltpu.SemaphoreType.DMA,
        ],
    )
    partials = run(patch_p, idx01, len_p)
    return jnp.sum(partials) / F


# speedup vs baseline: 1.0418x; 1.0019x over previous
import jax
import jax.numpy as jnp
from jax import lax
from jax.experimental import pallas as pl
from jax.experimental.pallas import tpu as pltpu
from jax.experimental.pallas import tpu_sc as plsc

F = 100000
E = 150000
D = 192
L = 16
NC, NS = 2, 16
NW = NC * NS
B = 128
C0, C1 = 38, 36
E_PAD = NS * B * (C0 + C1)
BT = 16384


def _relayout_kernel(xt_ref, o_ref):
    y = xt_ref[...].T.astype(jnp.bfloat16)
    a = y[:, :128]
    b = jnp.concatenate([y[:, 128:], jnp.zeros((BT, 64), jnp.bfloat16)],
                        axis=1)
    au = jax.lax.bitcast_convert_type(a, jnp.uint16).astype(jnp.int32)
    bu = jax.lax.bitcast_convert_type(b, jnp.uint16).astype(jnp.int32)
    o_ref[...] = au | (bu << 16)


def _tv_kernel(patch_hbm, idx01_hbm, len_hbm, out_hbm,
               i01a, lna, f1a, f2a,
               i01b, lnb, f1b, f2b,
               acc_v, gsema, gsemb, isem):
    cid = lax.axis_index("c")
    sid = lax.axis_index("s")
    phases = ((i01a, lna, f1a, f2a, gsema),
              (i01b, lnb, f1b, f2b, gsemb))

    def load_idx(base, c, ph):
        i01, ln, _, _, _ = ph
        off = pl.multiple_of(base + c * B, B)
        cp0 = pltpu.async_copy(idx01_hbm.at[:, pl.ds(off, B)], i01, isem)
        cp1 = pltpu.async_copy(len_hbm.at[pl.ds(off, B)], ln, isem)
        cp0.wait()
        cp1.wait()

    def fire_gathers(patch_hbm, ph):
        i01, _, f1, f2, gsem = ph
        pltpu.async_copy(patch_hbm.at[i01.at[0]], f1, gsem)
        pltpu.async_copy(patch_hbm.at[i01.at[1]], f2, gsem)

    def wait_gathers(patch_hbm, ph):
        i01, _, f1, f2, gsem = ph
        pltpu.make_async_copy(patch_hbm.at[i01.at[0]], f1, gsem).wait()
        pltpu.make_async_copy(patch_hbm.at[i01.at[1]], f2, gsem).wait()

    def compute(ph, tot):
        _, ln, f1, f2, _ = ph

        def group_body(g, t):
            w_blk = ln[pl.ds(g * L, L)]
            for k in range(L):
                e = g * L + k
                facc = None
                for h in range(128 // L):
                    x1 = plsc.bitcast(f1[e, pl.ds(h * L, L)], jnp.bfloat16)
                    x2 = plsc.bitcast(f2[e, pl.ds(h * L, L)], jnp.bfloat16)
                    d = x1 - x2
                    a = jnp.maximum(d, -d)
                    a0, a1 = plsc.unpack(
                        a, format=plsc.PackFormat.INTERLEAVED)
                    facc = a0 + a1 if facc is None else facc + a0 + a1
                t = t + w_blk[k] * facc
            return t

        return lax.fori_loop(0, B // L, group_body, tot)

    def run_core(patch_hbm, nchunks, base):
        base = pl.multiple_of(base, B)
        for b in range(2):
            load_idx(base, b, phases[b])
            fire_gathers(patch_hbm, phases[b])

        def pair_body(c2, tot):
            for b in range(2):
                ph = phases[b]
                c = 2 * c2 + b
                wait_gathers(patch_hbm, ph)
                tot = compute(ph, tot)

                @pl.when(c + 2 < nchunks)
                def _():
                    load_idx(base, c + 2, ph)
                    fire_gathers(patch_hbm, ph)

            return tot

        tot = lax.fori_loop(0, nchunks // 2, pair_body,
                            jnp.zeros((L,), jnp.float32))
        acc_v[...] = tot

    @pl.when(cid == 0)
    def _():
        run_core(patch_hbm, C0, sid * (B * C0))

    @pl.when(cid == 1)
    def _():
        run_core(patch_hbm, C1, NS * (B * C0) + sid * (B * C1))

    wid = sid * NC + cid
    pltpu.sync_copy(acc_v, out_hbm.at[pl.ds(wid * L, L)])


@jax.jit
def kernel(adv_patch, face_to_edges_idx, edge_len):
    patch_t = adv_patch.reshape(F, D).T
    patch_p = pl.pallas_call(
        _relayout_kernel,
        grid=(pl.cdiv(F, BT),),
        compiler_params=pltpu.CompilerParams(
            dimension_semantics=("parallel",)),
        in_specs=[pl.BlockSpec((D, BT), lambda i: (0, i))],
        out_specs=pl.BlockSpec((BT, 128), lambda i: (i, 0)),
        out_shape=jax.ShapeDtypeStruct((F, 128), jnp.int32),
    )(patch_t)

    idx_t = face_to_edges_idx.astype(jnp.int32).T
    pad = E_PAD - E
    fill = jnp.arange(pad, dtype=jnp.int32)
    idx01 = jnp.concatenate(
        [idx_t, jnp.stack([fill, fill])], axis=1)
    len_p = jnp.pad(edge_len, (0, pad))

    mesh = plsc.VectorSubcoreMesh(core_axis_name="c", subcore_axis_name="s")
    run = pl.kernel(
        _tv_kernel,
        mesh=mesh,
        compiler_params=pltpu.CompilerParams(use_tc_tiling_on_sc=True,
                                             needs_layout_passes=False),
        out_type=jax.ShapeDtypeStruct((NW * L,), jnp.float32),
        scratch_types=[
            pltpu.VMEM((2, B), jnp.int32),
            pltpu.VMEM((B,), jnp.float32),
            pltpu.VMEM((B, 128), jnp.int32),
            pltpu.VMEM((B, 128), jnp.int32),
            pltpu.VMEM((2, B), jnp.int32),
            pltpu.VMEM((B,), jnp.float32),
            pltpu.VMEM((B, 128), jnp.int32),
            pltpu.VMEM((B, 128), jnp.int32),
            pltpu.VMEM((L,), jnp.float32),
            pltpu.SemaphoreType.DMA,
            pltpu.SemaphoreType.DMA,
            pltpu.SemaphoreType.DMA,
        ],
    )
    partials = run(patch_p, idx01, len_p)
    return jnp.sum(partials) / F
